# Initial kernel scaffold; baseline (speedup 1.0000x reference)
import jax
import jax.numpy as jnp
from jax.experimental import pallas as pl


def kernel(f_query, f_kv, ln_q_g, ln_q_b, ln_kv_g, ln_kv_b, Wq, bq, W1, b1, W2, b2, Wa, ba, Wo, bo):
    raise NotImplementedError("write your pallas kernel here")


# speedup vs baseline: 1.2599x; 1.2599x over previous
import jax
import jax.numpy as jnp
from jax import lax
from jax.experimental import pallas as pl
from jax.experimental.pallas import tpu as pltpu

B, C, H, W, D = 2, 192, 32, 32, 16
NH, NP = 8, 4
HD = C // NH
N = H * W * D
T = B * N
NQ = NH * NP * 8
TM = 512


def _frontend_body(x_ref, g_ref, bg_ref, wqT_ref, bq_ref, w1T_ref, b1_ref,
                   w2pT_ref, b2p_ref, waT_ref, ba_ref, msk_ref,
                   idx_ref, wgt_ref):
    X = x_ref[...]
    m = jnp.mean(X, axis=1, keepdims=True)
    xc = X - m
    v = jnp.mean(xc * xc, axis=1, keepdims=True)
    Xn = xc * lax.rsqrt(v + 1e-5) * g_ref[...] + bg_ref[...]
    Q = jnp.dot(Xn, wqT_ref[...], preferred_element_type=jnp.float32) + bq_ref[...]
    Hd = jnp.maximum(jnp.dot(Q, w1T_ref[...], preferred_element_type=jnp.float32) + b1_ref[...], 0.0)
    offp = jnp.clip(jnp.dot(Hd, w2pT_ref[...], preferred_element_type=jnp.float32) + b2p_ref[...], -3.0, 3.0)
    A = jnp.dot(Q, waT_ref[...], preferred_element_type=jnp.float32) + ba_ref[...]
    A = A - jnp.max(A, axis=1, keepdims=True)
    E = jnp.exp(A)
    den = jnp.dot(E, msk_ref[...], preferred_element_type=jnp.float32)
    aw = E / den

    pid = pl.program_id(0)
    rid = pid * TM + lax.broadcasted_iota(jnp.int32, (TM, 1), 0)
    bidx = rid // N
    n = rid - bidx * N
    gh = n // (W * D)
    gw = (n // D) % W
    gd = n % D
    offx = offp[:, 0:32]
    offy = offp[:, 32:64]
    offz = offp[:, 64:96]
    x = jnp.clip(gh.astype(jnp.float32) + offx, 0.0, W - 1.0)
    y = jnp.clip(gw.astype(jnp.float32) + offy, 0.0, H - 1.0)
    z = jnp.clip(gd.astype(jnp.float32) + offz, 0.0, D - 1.0)
    x0f = jnp.floor(x); y0f = jnp.floor(y); z0f = jnp.floor(z)
    wx = x - x0f; wy = y - y0f; wz = z - z0f
    x0 = x0f.astype(jnp.int32); x1 = jnp.minimum(x0 + 1, W - 1)
    y0 = y0f.astype(jnp.int32); y1 = jnp.minimum(y0 + 1, H - 1)
    z0 = z0f.astype(jnp.int32); z1 = jnp.minimum(z0 + 1, D - 1)
    hlane = lax.broadcasted_iota(jnp.int32, (TM, 32), 1) // NP
    base = bidx * (NH * N) + hlane * N
    idx_parts = []
    wgt_parts = []
    for cz, (zi, wzc) in enumerate(((z0, 1.0 - wz), (z1, wz))):
        for cy, (yi, wyc) in enumerate(((y0, 1.0 - wy), (y1, wy))):
            for cx, (xi, wxc) in enumerate(((x0, 1.0 - wx), (x1, wx))):
                r = (yi * W + xi) * D + zi
                idx_parts.append(base + r)
                wgt_parts.append(aw * wzc * wyc * wxc)
    idx_ref[...] = jnp.concatenate(idx_parts, axis=1)
    wgt_ref[...] = jnp.concatenate(wgt_parts, axis=1)


def _proj_body(o_ref, woT_ref, bo_ref, out_ref):
    out_ref[...] = jnp.dot(o_ref[...], woT_ref[...], preferred_element_type=jnp.float32) + bo_ref[...]


def _row(v):
    return v.reshape(1, -1)


def kernel(f_query, f_kv, ln_q_g, ln_q_b, ln_kv_g, ln_kv_b, Wq, bq, W1, b1, W2, b2, Wa, ba, Wo, bo):
    X = f_query.reshape(B, C, N).transpose(0, 2, 1).reshape(T, C)
    W2pT = W2.T.reshape(C, NH * NP, 3).transpose(0, 2, 1).reshape(C, NH * NP * 3)
    b2p = b2.reshape(NH * NP, 3).T.reshape(-1)
    jj = jnp.arange(NH * NP)
    msk = (jj[:, None] // NP == jj[None, :] // NP).astype(jnp.float32)

    full = lambda s: pl.BlockSpec(s, lambda i: (0, 0))
    idx, wgt = pl.pallas_call(
        _frontend_body,
        grid=(T // TM,),
        in_specs=[
            pl.BlockSpec((TM, C), lambda i: (i, 0)),
            full((1, C)), full((1, C)),
            full((C, C)), full((1, C)),
            full((C, C)), full((1, C)),
            full((C, NH * NP * 3)), full((1, NH * NP * 3)),
            full((C, NH * NP)), full((1, NH * NP)),
            full((NH * NP, NH * NP)),
        ],
        out_specs=[
            pl.BlockSpec((TM, NQ), lambda i: (i, 0)),
            pl.BlockSpec((TM, NQ), lambda i: (i, 0)),
        ],
        out_shape=[
            jax.ShapeDtypeStruct((T, NQ), jnp.int32),
            jax.ShapeDtypeStruct((T, NQ), jnp.float32),
        ],
    )(X, _row(ln_q_g), _row(ln_q_b), Wq.T, _row(bq), W1.T, _row(b1),
      W2pT, _row(b2p), Wa.T, _row(ba), msk)

    G = f_kv.reshape(B, NH, HD, H, W, D).transpose(0, 1, 3, 4, 5, 2).reshape(B * NH * N, HD)

    rows = G[idx.reshape(-1)].reshape(T, 8, NH, NP, HD)
    O = jnp.einsum('tchp,tchpd->thd', wgt.reshape(T, 8, NH, NP), rows).reshape(T, C)

    out = pl.pallas_call(
        _proj_body,
        grid=(T // TM,),
        in_specs=[
            pl.BlockSpec((TM, C), lambda i: (i, 0)),
            full((C, C)), full((1, C)),
        ],
        out_specs=pl.BlockSpec((TM, C), lambda i: (i, 0)),
        out_shape=jax.ShapeDtypeStruct((T, C), jnp.float32),
    )(O, Wo.T, _row(bo))

    return out.reshape(B, N, C).transpose(0, 2, 1).reshape(B, C, H, W, D)


# speedup vs baseline: 113.4065x; 90.0121x over previous
import functools

import jax
import jax.numpy as jnp
from jax import lax
from jax.experimental import pallas as pl
from jax.experimental.pallas import tpu as pltpu
from jax.experimental.pallas import tpu_sc as plsc

B, C, H, W, D = 2, 192, 32, 32, 16
NH, NP = 8, 4
HD = C // NH
N = H * W * D
T = B * N
NQ = NH * NP * 8
TM = 512


def _frontend_body(x_ref, g_ref, bg_ref, wqT_ref, bq_ref, w1T_ref, b1_ref,
                   w2pT_ref, b2p_ref, waT_ref, ba_ref, msk_ref,
                   idx_ref, wgt_ref):
    X = x_ref[...]
    m = jnp.mean(X, axis=1, keepdims=True)
    xc = X - m
    v = jnp.mean(xc * xc, axis=1, keepdims=True)
    Xn = xc * lax.rsqrt(v + 1e-5) * g_ref[...] + bg_ref[...]
    Q = jnp.dot(Xn, wqT_ref[...], preferred_element_type=jnp.float32) + bq_ref[...]
    Hd = jnp.maximum(jnp.dot(Q, w1T_ref[...], preferred_element_type=jnp.float32) + b1_ref[...], 0.0)
    offp = jnp.clip(jnp.dot(Hd, w2pT_ref[...], preferred_element_type=jnp.float32) + b2p_ref[...], -3.0, 3.0)
    A = jnp.dot(Q, waT_ref[...], preferred_element_type=jnp.float32) + ba_ref[...]
    A = A - jnp.max(A, axis=1, keepdims=True)
    E = jnp.exp(A)
    den = jnp.dot(E, msk_ref[...], preferred_element_type=jnp.float32)
    aw = E / den

    pid = pl.program_id(0)
    rid = pid * TM + lax.broadcasted_iota(jnp.int32, (TM, 1), 0)
    bidx = rid // N
    n = rid - bidx * N
    gh = n // (W * D)
    gw = (n // D) % W
    gd = n % D
    offx = offp[:, 0:32]
    offy = offp[:, 32:64]
    offz = offp[:, 64:96]
    x = jnp.clip(gh.astype(jnp.float32) + offx, 0.0, W - 1.0)
    y = jnp.clip(gw.astype(jnp.float32) + offy, 0.0, H - 1.0)
    z = jnp.clip(gd.astype(jnp.float32) + offz, 0.0, D - 1.0)
    x0f = jnp.floor(x); y0f = jnp.floor(y); z0f = jnp.floor(z)
    wx = x - x0f; wy = y - y0f; wz = z - z0f
    x0 = x0f.astype(jnp.int32); x1 = jnp.minimum(x0 + 1, W - 1)
    y0 = y0f.astype(jnp.int32); y1 = jnp.minimum(y0 + 1, H - 1)
    z0 = z0f.astype(jnp.int32); z1 = jnp.minimum(z0 + 1, D - 1)
    hlane = lax.broadcasted_iota(jnp.int32, (TM, 32), 1) // NP
    base = bidx * (NH * N) + hlane * N
    idx_parts = []
    wgt_parts = []
    for cz, (zi, wzc) in enumerate(((z0, 1.0 - wz), (z1, wz))):
        for cy, (yi, wyc) in enumerate(((y0, 1.0 - wy), (y1, wy))):
            for cx, (xi, wxc) in enumerate(((x0, 1.0 - wx), (x1, wx))):
                r = (yi * W + xi) * D + zi
                idx_parts.append(base + r)
                wgt_parts.append(aw * wzc * wyc * wxc)
    idx_ref[...] = jnp.concatenate(idx_parts, axis=1)
    wgt_ref[...] = jnp.concatenate(wgt_parts, axis=1)


NW = 32
TPW = T // NW
KT = 4
NCH = TPW // KT
ROWS = KT * NQ
IR = ROWS // 128
HDP = 32
CP = NH * HDP


def _sc_gather_body(g_ref, idx_ref, wgt_ref, out_ref,
                    idxb, wgtb, rowb, outb, sem0, sem1):
    sems = (sem0, sem1)
    wid = lax.axis_index("s") * 2 + lax.axis_index("c")
    tok0 = wid * TPW

    def fill(slot, ch):
        row0 = (tok0 + ch * KT) * (NQ // 128)
        pltpu.sync_copy(idx_ref.at[pl.ds(row0, IR)], idxb.at[slot])
        pltpu.sync_copy(wgt_ref.at[pl.ds(row0 * 128, ROWS)], wgtb.at[slot])
        for j in range(IR):
            pltpu.async_copy(g_ref.at[idxb.at[slot, j]],
                             rowb.at[slot, pl.ds(j * 128, 128)], sems[slot])

    def drain(slot):
        pltpu.make_async_copy(g_ref.at[pl.ds(0, ROWS)], rowb.at[slot],
                              sems[slot]).wait()

    def compute(slot, ch):
        def token_body(t, _):
            for h in range(NH):
                acc0 = jnp.zeros((16,), jnp.float32)
                acc1 = jnp.zeros((16,), jnp.float32)
                for c8 in range(8):
                    wv16 = wgtb[slot, pl.ds(t * NQ + c8 * 32 + (h // 4) * 16, 16)]
                    for p in range(NP):
                        q = c8 * 32 + h * NP + p
                        wv = jnp.full((16,), wv16[(h % 4) * NP + p], jnp.float32)
                        r = t * NQ + q
                        acc0 = acc0 + wv * rowb[slot, r, pl.ds(0, 16)]
                        acc1 = acc1 + wv * rowb[slot, r, pl.ds(16, 16)]
                o = t * CP + h * HDP
                outb[slot, pl.ds(o, 16)] = acc0
                outb[slot, pl.ds(o + 16, 16)] = acc1
            return 0
        lax.fori_loop(0, KT, token_body, 0)
        obase = (tok0 + ch * KT) * CP
        pltpu.sync_copy(outb.at[slot], out_ref.at[pl.ds(obase, KT * CP)])

    fill(0, 0)
    fill(1, 1)

    def chunk_body(i, _):
        g = i * 2
        for slot in range(2):
            ch = g + slot
            drain(slot)
            compute(slot, ch)

            @pl.when(ch + 2 < NCH)
            def _():
                fill(slot, ch + 2)
        return 0

    lax.fori_loop(0, NCH // 2, chunk_body, 0)


@functools.partial(
    pl.kernel,
    out_type=jax.ShapeDtypeStruct((T * CP,), jnp.float32),
    mesh=plsc.VectorSubcoreMesh(core_axis_name="c", subcore_axis_name="s"),
    compiler_params=pltpu.CompilerParams(use_tc_tiling_on_sc=False),
    scratch_types=[
        pltpu.VMEM((2, IR, 128), jnp.int32),
        pltpu.VMEM((2, ROWS), jnp.float32),
        pltpu.VMEM((2, ROWS, HDP), jnp.float32),
        pltpu.VMEM((2, KT * CP), jnp.float32),
        pltpu.SemaphoreType.DMA,
        pltpu.SemaphoreType.DMA,
    ],
)
def _sc_gather(g_ref, idx_ref, wgt_ref, out_ref, idxb, wgtb, rowb, outb, sem0, sem1):
    _sc_gather_body(g_ref, idx_ref, wgt_ref, out_ref,
                    idxb, wgtb, rowb, outb, sem0, sem1)


def _proj_body(o_ref, woT_ref, bo_ref, out_ref):
    out_ref[...] = jnp.dot(o_ref[...], woT_ref[...], preferred_element_type=jnp.float32) + bo_ref[...]


def _row(v):
    return v.reshape(1, -1)


def kernel(f_query, f_kv, ln_q_g, ln_q_b, ln_kv_g, ln_kv_b, Wq, bq, W1, b1, W2, b2, Wa, ba, Wo, bo):
    X = f_query.reshape(B, C, N).transpose(0, 2, 1).reshape(T, C)
    W2pT = W2.T.reshape(C, NH * NP, 3).transpose(0, 2, 1).reshape(C, NH * NP * 3)
    b2p = b2.reshape(NH * NP, 3).T.reshape(-1)
    jj = jnp.arange(NH * NP)
    msk = (jj[:, None] // NP == jj[None, :] // NP).astype(jnp.float32)

    full = lambda s: pl.BlockSpec(s, lambda i: (0, 0))
    idx, wgt = pl.pallas_call(
        _frontend_body,
        grid=(T // TM,),
        in_specs=[
            pl.BlockSpec((TM, C), lambda i: (i, 0)),
            full((1, C)), full((1, C)),
            full((C, C)), full((1, C)),
            full((C, C)), full((1, C)),
            full((C, NH * NP * 3)), full((1, NH * NP * 3)),
            full((C, NH * NP)), full((1, NH * NP)),
            full((NH * NP, NH * NP)),
        ],
        out_specs=[
            pl.BlockSpec((TM, NQ), lambda i: (i, 0)),
            pl.BlockSpec((TM, NQ), lambda i: (i, 0)),
        ],
        out_shape=[
            jax.ShapeDtypeStruct((T, NQ), jnp.int32),
            jax.ShapeDtypeStruct((T, NQ), jnp.float32),
        ],
    )(X, _row(ln_q_g), _row(ln_q_b), Wq.T, _row(bq), W1.T, _row(b1),
      W2pT, _row(b2p), Wa.T, _row(ba), msk)

    G = jnp.pad(
        f_kv.reshape(B, NH, HD, H, W, D).transpose(0, 1, 3, 4, 5, 2),
        ((0, 0), (0, 0), (0, 0), (0, 0), (0, 0), (0, HDP - HD)),
    ).reshape(B * NH * N, HDP)

    O = _sc_gather(G, idx.reshape(-1, 128), wgt.reshape(-1)).reshape(T, CP)

    WoP = jnp.pad(Wo.T.reshape(NH, HD, C), ((0, 0), (0, HDP - HD), (0, 0))).reshape(CP, C)
    out = pl.pallas_call(
        _proj_body,
        grid=(T // TM,),
        in_specs=[
            pl.BlockSpec((TM, CP), lambda i: (i, 0)),
            full((CP, C)), full((1, C)),
        ],
        out_specs=pl.BlockSpec((TM, C), lambda i: (i, 0)),
        out_shape=jax.ShapeDtypeStruct((T, C), jnp.float32),
    )(O, WoP, _row(bo))

    return out.reshape(B, N, C).transpose(0, 2, 1).reshape(B, C, H, W, D)
